# baseline (speedup 1.0000x reference)
import functools

import jax
import jax.numpy as jnp
from jax import lax
from jax.experimental import pallas as pl
from jax.experimental.pallas import tpu as pltpu
from jax.experimental.pallas import tpu_sc as plsc

BATCH = 16384
FIELDS = 26
DIM = 64
PAD_DIM = 128
VOCAB = 1000001
VOCAB_PAD = 1000008

NC = 2
NS = 16
NW = NC * NS
L = 16
B = BATCH * FIELDS
B_PER_W = B // NW
ROWS_PER_W = BATCH // NW
CHUNK_ROWS = 8
CHUNK = CHUNK_ROWS * FIELDS
N_CHUNKS = B_PER_W // CHUNK

TB = 128
TB_PER_W = 244
TB_EXTRA_BASE = NW * TB_PER_W * TB
TAIL_BASE = 999936
TAIL_N = 64

_mesh = plsc.VectorSubcoreMesh(core_axis_name="c", subcore_axis_name="s")


@functools.partial(
    pl.kernel,
    mesh=_mesh,
    out_type=(
        jax.ShapeDtypeStruct((VOCAB_PAD, PAD_DIM), jnp.float32),
        jax.ShapeDtypeStruct((B,), jnp.int32),
    ),
    scratch_types=[
        pltpu.VMEM((FIELDS, ROWS_PER_W), jnp.int32),
        pltpu.VMEM((B_PER_W,), jnp.int32),
        pltpu.VMEM((DIM, TB), jnp.float32),
        pltpu.VMEM((DIM, TB), jnp.float32),
        pltpu.VMEM((TB, PAD_DIM), jnp.float32),
        pltpu.VMEM((TB, PAD_DIM), jnp.float32),
        pltpu.VMEM((TAIL_N, DIM), jnp.float32),
        pltpu.SemaphoreType.DMA,
        pltpu.SemaphoreType.DMA,
        pltpu.SemaphoreType.DMA,
        pltpu.SemaphoreType.DMA,
    ],
    compiler_params=pltpu.CompilerParams(
        use_tc_tiling_on_sc=True, needs_layout_passes=False
    ),
)
def _prep_kernel(
    embt_hbm, xt_hbm, tbl_hbm, idx_hbm,
    xv, idxbuf, s0, s1, w0v, w1v, tailv, i0, i1, o0, o1,
):
    wid = lax.axis_index("s") * NC + lax.axis_index("c")
    lanes = lax.iota(jnp.int32, L)

    b0 = wid * ROWS_PER_W
    pltpu.sync_copy(xt_hbm.at[:, pl.ds(b0, ROWS_PER_W)], xv)

    def flatten_body(k, carry):
        f, b = carry
        v = plsc.load_gather(xv, [f, b])
        idxbuf[pl.ds(k * L, L)] = v
        f = f + L
        wrap = f >= FIELDS
        f = jnp.where(wrap, f - FIELDS, f)
        b = jnp.where(wrap, b + 1, b)
        return f, b

    lax.fori_loop(
        0, B_PER_W // L, flatten_body, (lanes, jnp.zeros(L, jnp.int32))
    )
    pltpu.sync_copy(idxbuf, idx_hbm.at[pl.ds(wid * B_PER_W, B_PER_W)])

    d16 = [lanes + k * L for k in range(DIM // L)]

    def transpose_block(stage_v, wide_v):
        def row_body(r4, carry):
            for dr in range(4):
                v = r4 * 4 + dr
                vv = jnp.zeros(L, jnp.int32) + v
                for k in range(DIM // L):
                    wide_v[v, pl.ds(k * L, L)] = plsc.load_gather(
                        stage_v, [d16[k], vv]
                    )
            return carry

        lax.fori_loop(0, TB // 4, row_body, 0)

    def start_in(stage_v, sem, v0):
        pltpu.async_copy(embt_hbm.at[:, pl.ds(v0, TB)], stage_v, sem)

    def wait_in(stage_v, sem):
        pltpu.make_async_copy(
            embt_hbm.at[:, pl.ds(0, TB)], stage_v, sem
        ).wait()

    def start_out(wide_v, sem, v0):
        pltpu.async_copy(wide_v, tbl_hbm.at[pl.ds(v0, TB)], sem)

    def wait_out(wide_v, sem):
        pltpu.make_async_copy(
            wide_v, tbl_hbm.at[pl.ds(0, TB)], sem
        ).wait()

    base = wid * TB_PER_W * TB
    start_in(s0, i0, base)

    def pair_body(j, carry):
        wait_in(s0, i0)
        start_in(s1, i1, base + (2 * j + 1) * TB)

        @pl.when(j > 0)
        def _():
            wait_out(w0v, o0)

        transpose_block(s0, w0v)
        start_out(w0v, o0, base + 2 * j * TB)
        wait_in(s1, i1)

        @pl.when(2 * j + 2 < TB_PER_W)
        def _():
            start_in(s0, i0, base + (2 * j + 2) * TB)

        @pl.when(j > 0)
        def _():
            wait_out(w1v, o1)

        transpose_block(s1, w1v)
        start_out(w1v, o1, base + (2 * j + 1) * TB)
        return carry

    lax.fori_loop(0, TB_PER_W // 2, pair_body, 0)
    wait_out(w0v, o0)
    wait_out(w1v, o1)

    @pl.when(wid < 4)
    def _extra():
        v0 = TB_EXTRA_BASE + wid * TB
        pltpu.sync_copy(embt_hbm.at[:, pl.ds(v0, TB)], s0)
        transpose_block(s0, w0v)
        pltpu.sync_copy(w0v, tbl_hbm.at[pl.ds(v0, TB)])

    @pl.when(wid == 4)
    def _tail():
        for d in range(DIM):
            pltpu.sync_copy(
                embt_hbm.at[d, pl.ds(TAIL_BASE, TAIL_N)], tailv.at[d]
            )

        def tail_row(v, carry):
            vv = jnp.zeros(L, jnp.int32) + v
            for k in range(DIM // L):
                w0v[v, pl.ds(k * L, L)] = plsc.load_gather(
                    tailv, [d16[k], vv]
                )
            return carry

        lax.fori_loop(0, TAIL_N, tail_row, 0)
        pltpu.sync_copy(
            w0v.at[pl.ds(0, TAIL_N)], tbl_hbm.at[pl.ds(TAIL_BASE, TAIL_N)]
        )


@functools.partial(
    pl.kernel,
    mesh=_mesh,
    out_type=jax.ShapeDtypeStruct((BATCH, FIELDS, DIM), jnp.float32),
    scratch_types=[
        pltpu.VMEM((B_PER_W,), jnp.int32),
        pltpu.VMEM((CHUNK, PAD_DIM), jnp.float32),
        pltpu.VMEM((CHUNK, PAD_DIM), jnp.float32),
        pltpu.SemaphoreType.DMA,
        pltpu.SemaphoreType.DMA,
        pltpu.SemaphoreType.DMA,
        pltpu.SemaphoreType.DMA,
    ],
    compiler_params=pltpu.CompilerParams(
        use_tc_tiling_on_sc=False, needs_layout_passes=False
    ),
)
def _gather_kernel(
    tbl_hbm, idx_hbm, out_hbm, idx_v, rows0_v, rows1_v, g0, g1, w0, w1
):
    wid = lax.axis_index("s") * NC + lax.axis_index("c")
    base = wid * B_PER_W
    row0 = wid * ROWS_PER_W
    pltpu.sync_copy(idx_hbm.at[pl.ds(base, B_PER_W)], idx_v)

    def gather(c, rows_v, gsem):
        pltpu.async_copy(
            tbl_hbm.at[idx_v.at[pl.ds(c * CHUNK, CHUNK)]], rows_v, gsem
        )

    def wait_gather(rows_v, gsem):
        pltpu.make_async_copy(
            tbl_hbm.at[pl.ds(0, CHUNK)], rows_v, gsem
        ).wait()

    def writeback(c, rows_v, wsem):
        for k in range(CHUNK_ROWS):
            pltpu.async_copy(
                rows_v.at[pl.ds(k * FIELDS, FIELDS), pl.ds(0, DIM)],
                out_hbm.at[row0 + c * CHUNK_ROWS + k],
                wsem,
            )

    def wait_writeback(rows_v, wsem):
        for k in range(CHUNK_ROWS):
            pltpu.make_async_copy(
                rows_v.at[pl.ds(0, FIELDS), pl.ds(0, DIM)],
                out_hbm.at[0],
                wsem,
            ).wait()

    slots = ((rows0_v, g0, w0), (rows1_v, g1, w1))
    gather(0, rows0_v, g0)

    def chunk_pair(j, carry):
        for s in range(2):
            c = j * 2 + s
            rows_v, gsem, wsem = slots[s]
            o_rows, o_g, o_w = slots[1 - s]

            @pl.when(c > 0)
            def _():
                wait_writeback(o_rows, o_w)

            @pl.when(c + 1 < N_CHUNKS)
            def _():
                gather(c + 1, o_rows, o_g)

            wait_gather(rows_v, gsem)
            writeback(c, rows_v, wsem)
        return carry

    lax.fori_loop(0, N_CHUNKS // 2, chunk_pair, 0)
    wait_writeback(rows1_v, w1)


def kernel(x, emb):
    tbl, idx = _prep_kernel(emb.T, x.astype(jnp.int32).T)
    return _gather_kernel(tbl, idx)


# speedup vs baseline: 1.9868x; 1.9868x over previous
import functools

import jax
import jax.numpy as jnp
from jax import lax
from jax.experimental import pallas as pl
from jax.experimental.pallas import tpu as pltpu
from jax.experimental.pallas import tpu_sc as plsc

BATCH = 16384
FIELDS = 26
DIM = 64
PAD_DIM = 128
VOCAB = 1000001
VOCAB_PAD = 1000008

NC = 2
NS = 16
NW = NC * NS
L = 16
B = BATCH * FIELDS
B_PER_W = B // NW
ROWS_PER_W = BATCH // NW
CHUNK_ROWS = 8
CHUNK = CHUNK_ROWS * FIELDS
N_CHUNKS = B_PER_W // CHUNK

TB = 128
TB_PER_W = 244
TB_EXTRA_BASE = NW * TB_PER_W * TB
TAIL_BASE = 999936
TAIL_N = 64

_mesh = plsc.VectorSubcoreMesh(core_axis_name="c", subcore_axis_name="s")


@functools.partial(
    pl.kernel,
    mesh=_mesh,
    out_type=(
        jax.ShapeDtypeStruct((VOCAB_PAD, PAD_DIM), jnp.float32),
        jax.ShapeDtypeStruct((B,), jnp.int32),
    ),
    scratch_types=[
        pltpu.VMEM((FIELDS, ROWS_PER_W), jnp.int32),
        pltpu.VMEM((B_PER_W,), jnp.int32),
        pltpu.VMEM((DIM, TB), jnp.float32),
        pltpu.VMEM((DIM, TB), jnp.float32),
        pltpu.VMEM((TB, PAD_DIM), jnp.float32),
        pltpu.VMEM((TB, PAD_DIM), jnp.float32),
        pltpu.VMEM((TAIL_N, DIM), jnp.float32),
        pltpu.SemaphoreType.DMA,
        pltpu.SemaphoreType.DMA,
        pltpu.SemaphoreType.DMA,
        pltpu.SemaphoreType.DMA,
    ],
    compiler_params=pltpu.CompilerParams(
        use_tc_tiling_on_sc=True, needs_layout_passes=False
    ),
)
def _prep_kernel(
    embt_hbm, xt_hbm, tbl_hbm, idx_hbm,
    xv, idxbuf, s0, s1, w0v, w1v, tailv, i0, i1, o0, o1,
):
    wid = lax.axis_index("s") * NC + lax.axis_index("c")
    lanes = lax.iota(jnp.int32, L)

    b0 = wid * ROWS_PER_W
    pltpu.sync_copy(xt_hbm.at[:, pl.ds(b0, ROWS_PER_W)], xv)

    def flatten_body(k, carry):
        f, b = carry
        v = plsc.load_gather(xv, [f, b])
        idxbuf[pl.ds(k * L, L)] = v
        f = f + L
        wrap = f >= FIELDS
        f = jnp.where(wrap, f - FIELDS, f)
        b = jnp.where(wrap, b + 1, b)
        return f, b

    lax.fori_loop(
        0, B_PER_W // L, flatten_body, (lanes, jnp.zeros(L, jnp.int32))
    )
    pltpu.sync_copy(idxbuf, idx_hbm.at[pl.ds(wid * B_PER_W, B_PER_W)])

    d16 = [lanes + k * L for k in range(DIM // L)]

    def transpose_block(stage_v, wide_v):
        def tile_body(tv, carry):
            v0 = tv * L
            for kd in range(DIM // L):
                dd = d16[kd]
                for t in range(L):
                    vs = v0 + ((lanes + t) & (L - 1))
                    vec = plsc.load_gather(stage_v, [dd, vs])
                    plsc.store_scatter(wide_v, [vs, dd], vec)
            return carry

        lax.fori_loop(0, TB // L, tile_body, 0)

    def start_in(stage_v, sem, v0):
        pltpu.async_copy(embt_hbm.at[:, pl.ds(v0, TB)], stage_v, sem)

    def wait_in(stage_v, sem):
        pltpu.make_async_copy(
            embt_hbm.at[:, pl.ds(0, TB)], stage_v, sem
        ).wait()

    def start_out(wide_v, sem, v0):
        pltpu.async_copy(wide_v, tbl_hbm.at[pl.ds(v0, TB)], sem)

    def wait_out(wide_v, sem):
        pltpu.make_async_copy(
            wide_v, tbl_hbm.at[pl.ds(0, TB)], sem
        ).wait()

    base = wid * TB_PER_W * TB
    start_in(s0, i0, base)

    def pair_body(j, carry):
        wait_in(s0, i0)
        start_in(s1, i1, base + (2 * j + 1) * TB)

        @pl.when(j > 0)
        def _():
            wait_out(w0v, o0)

        transpose_block(s0, w0v)
        start_out(w0v, o0, base + 2 * j * TB)
        wait_in(s1, i1)

        @pl.when(2 * j + 2 < TB_PER_W)
        def _():
            start_in(s0, i0, base + (2 * j + 2) * TB)

        @pl.when(j > 0)
        def _():
            wait_out(w1v, o1)

        transpose_block(s1, w1v)
        start_out(w1v, o1, base + (2 * j + 1) * TB)
        return carry

    lax.fori_loop(0, TB_PER_W // 2, pair_body, 0)
    wait_out(w0v, o0)
    wait_out(w1v, o1)

    @pl.when(wid < 4)
    def _extra():
        v0 = TB_EXTRA_BASE + wid * TB
        pltpu.sync_copy(embt_hbm.at[:, pl.ds(v0, TB)], s0)
        transpose_block(s0, w0v)
        pltpu.sync_copy(w0v, tbl_hbm.at[pl.ds(v0, TB)])

    @pl.when(wid == 4)
    def _tail():
        for d in range(DIM):
            pltpu.sync_copy(
                embt_hbm.at[d, pl.ds(TAIL_BASE, TAIL_N)], tailv.at[d]
            )

        def tail_row(v, carry):
            vv = jnp.zeros(L, jnp.int32) + v
            for k in range(DIM // L):
                w0v[v, pl.ds(k * L, L)] = plsc.load_gather(
                    tailv, [d16[k], vv]
                )
            return carry

        lax.fori_loop(0, TAIL_N, tail_row, 0)
        pltpu.sync_copy(
            w0v.at[pl.ds(0, TAIL_N)], tbl_hbm.at[pl.ds(TAIL_BASE, TAIL_N)]
        )


@functools.partial(
    pl.kernel,
    mesh=_mesh,
    out_type=jax.ShapeDtypeStruct((BATCH, FIELDS, DIM), jnp.float32),
    scratch_types=[
        pltpu.VMEM((B_PER_W,), jnp.int32),
        pltpu.VMEM((CHUNK, PAD_DIM), jnp.float32),
        pltpu.VMEM((CHUNK, PAD_DIM), jnp.float32),
        pltpu.SemaphoreType.DMA,
        pltpu.SemaphoreType.DMA,
        pltpu.SemaphoreType.DMA,
        pltpu.SemaphoreType.DMA,
    ],
    compiler_params=pltpu.CompilerParams(
        use_tc_tiling_on_sc=False, needs_layout_passes=False
    ),
)
def _gather_kernel(
    tbl_hbm, idx_hbm, out_hbm, idx_v, rows0_v, rows1_v, g0, g1, w0, w1
):
    wid = lax.axis_index("s") * NC + lax.axis_index("c")
    base = wid * B_PER_W
    row0 = wid * ROWS_PER_W
    pltpu.sync_copy(idx_hbm.at[pl.ds(base, B_PER_W)], idx_v)

    def gather(c, rows_v, gsem):
        pltpu.async_copy(
            tbl_hbm.at[idx_v.at[pl.ds(c * CHUNK, CHUNK)]], rows_v, gsem
        )

    def wait_gather(rows_v, gsem):
        pltpu.make_async_copy(
            tbl_hbm.at[pl.ds(0, CHUNK)], rows_v, gsem
        ).wait()

    def writeback(c, rows_v, wsem):
        for k in range(CHUNK_ROWS):
            pltpu.async_copy(
                rows_v.at[pl.ds(k * FIELDS, FIELDS), pl.ds(0, DIM)],
                out_hbm.at[row0 + c * CHUNK_ROWS + k],
                wsem,
            )

    def wait_writeback(rows_v, wsem):
        for k in range(CHUNK_ROWS):
            pltpu.make_async_copy(
                rows_v.at[pl.ds(0, FIELDS), pl.ds(0, DIM)],
                out_hbm.at[0],
                wsem,
            ).wait()

    slots = ((rows0_v, g0, w0), (rows1_v, g1, w1))
    gather(0, rows0_v, g0)

    def chunk_pair(j, carry):
        for s in range(2):
            c = j * 2 + s
            rows_v, gsem, wsem = slots[s]
            o_rows, o_g, o_w = slots[1 - s]

            @pl.when(c > 0)
            def _():
                wait_writeback(o_rows, o_w)

            @pl.when(c + 1 < N_CHUNKS)
            def _():
                gather(c + 1, o_rows, o_g)

            wait_gather(rows_v, gsem)
            writeback(c, rows_v, wsem)
        return carry

    lax.fori_loop(0, N_CHUNKS // 2, chunk_pair, 0)
    wait_writeback(rows1_v, w1)


def kernel(x, emb):
    tbl, idx = _prep_kernel(emb.T, x.astype(jnp.int32).T)
    return _gather_kernel(tbl, idx)
